# Initial kernel scaffold; baseline (speedup 1.0000x reference)
import jax
import jax.numpy as jnp
from jax.experimental import pallas as pl


def kernel(x, edge_index, edge_weight, params):
    raise NotImplementedError("write your pallas kernel here")


# speedup vs baseline: 33.5155x; 33.5155x over previous
import functools
import jax
import jax.numpy as jnp
from jax import lax
from jax.experimental import pallas as pl
from jax.experimental.pallas import tpu as pltpu
from jax.experimental.pallas import tpu_sc as plsc

N = 10000
E = 320000
NC = 2
NS = 16
NW = NC * NS
EPW = E // NW
CHUNK = 80
KCH = EPW // CHUNK
ROWS_PER = 624
ROWS_TAIL = N - NS * ROWS_PER


def _sc_aggregate(table, rows3, cols3, width):
    mesh = plsc.VectorSubcoreMesh(core_axis_name="c", subcore_axis_name="s")
    zeros = jnp.zeros((N, width), jnp.float32)

    @functools.partial(
        pl.kernel,
        mesh=mesh,
        out_type=jax.ShapeDtypeStruct((NC, N, width), jnp.float32),
        compiler_params=pltpu.CompilerParams(use_tc_tiling_on_sc=False),
        scratch_types=[
            pltpu.VMEM((KCH, CHUNK), jnp.int32),
            pltpu.VMEM((KCH, CHUNK), jnp.int32),
            pltpu.VMEM((CHUNK, width), jnp.float32),
            pltpu.VMEM((CHUNK, width), jnp.float32),
            pltpu.VMEM_SHARED((N, width), jnp.float32),
            pltpu.SemaphoreType.DMA,
            pltpu.SemaphoreType.DMA,
        ],
    )
    def k(table_hbm, rows_hbm, cols_hbm, zeros_hbm, out_hbm,
          rowv, colv, gbuf0, gbuf1, acc, sem0, sem1):
        cid = lax.axis_index("c")
        sid = lax.axis_index("s")
        wid = sid * NC + cid

        pltpu.sync_copy(zeros_hbm.at[pl.ds(sid * ROWS_PER, ROWS_PER)],
                        acc.at[pl.ds(sid * ROWS_PER, ROWS_PER)])

        @pl.when(sid == 0)
        def _():
            pltpu.sync_copy(zeros_hbm.at[pl.ds(NS * ROWS_PER, ROWS_TAIL)],
                            acc.at[pl.ds(NS * ROWS_PER, ROWS_TAIL)])
        pltpu.sync_copy(rows_hbm.at[wid], rowv)
        pltpu.sync_copy(cols_hbm.at[wid], colv)
        plsc.subcore_barrier()

        gbufs = (gbuf0, gbuf1)
        sems = (sem0, sem1)
        cp0 = pltpu.async_copy(table_hbm.at[rowv.at[0]], gbuf0, sem0)

        def body(j, _):
            @pl.when(j + 1 < KCH)
            def _():
                for b in range(2):
                    @pl.when(lax.rem(j + 1, 2) == b)
                    def _():
                        pltpu.async_copy(table_hbm.at[rowv.at[j + 1]],
                                         gbufs[b], sems[b])
            for b in range(2):
                @pl.when(lax.rem(j, 2) == b)
                def _():
                    pltpu.make_async_copy(table_hbm.at[rowv.at[j]],
                                          gbufs[b], sems[b]).wait()
                    pltpu.sync_copy(gbufs[b], acc.at[colv.at[j]], add=True)
            return 0

        lax.fori_loop(0, KCH, body, 0, unroll=False)
        del cp0
        plsc.subcore_barrier()
        pltpu.sync_copy(acc.at[pl.ds(sid * ROWS_PER, ROWS_PER)],
                        out_hbm.at[cid, pl.ds(sid * ROWS_PER, ROWS_PER)])

        @pl.when(sid == 0)
        def _():
            pltpu.sync_copy(acc.at[pl.ds(NS * ROWS_PER, ROWS_TAIL)],
                            out_hbm.at[cid, pl.ds(NS * ROWS_PER, ROWS_TAIL)])

    return k(table, rows3, cols3, zeros)


def _dot(a, b):
    return jnp.dot(a.astype(jnp.bfloat16), b.astype(jnp.bfloat16),
                   preferred_element_type=jnp.float32)


def _rcp(q):
    r = 1.0 / q
    r = r * (2.0 - q * r)
    r = r * (2.0 - q * r)
    return r


def _rsqrt(v):
    y = lax.rsqrt(v)
    y = y * (1.5 - 0.5 * v * y * y)
    y = y * (1.5 - 0.5 * v * y * y)
    return y


def _tanh(x):
    x = jnp.clip(x, -9.0, 9.0)
    x2 = x * x
    p = x * (4.89352455891786e-03 + x2 * (6.37261928875436e-04 + x2 * (
        1.48572235717979e-05 + x2 * (5.12229709037114e-08 + x2 * (
            -8.60467152213735e-11 + x2 * (2.00018790482477e-13 + x2 * (
                -2.76076847742355e-16)))))))
    q = 4.89352518554385e-03 + x2 * (2.26843463243900e-03 + x2 * (
        1.18534705686654e-04 + x2 * 1.19825839466702e-06))
    return p * _rcp(q)


def _sigmoid(x):
    return 0.5 + 0.5 * _tanh(0.5 * x)


def _gates(h, p):
    I = _sigmoid(_dot(h, p['W_i'][...]) + p['bi'][...])
    T = _tanh(_dot(h, p['W_c'][...]) + p['bc'][...])
    C = I * T
    O = _sigmoid(_dot(h, p['W_o'][...]) + p['wco'][...] * C + p['bo'][...])
    H = O * _tanh(C)
    return _dot(H, p['Wg'][...])


R = 2000
NB = N // R


def _full(shape):
    return pl.BlockSpec(shape, lambda i: tuple(0 for _ in shape))


def _tc_first(x, dinv, lw):
    names = ['W_i', 'W_c', 'W_o', 'bi', 'bc', 'bo', 'wco', 'Wg']

    def body(x_ref, dinv_ref, *rest):
        refs = dict(zip(names, rest[:8]))
        h3_out = rest[8]
        h2 = _gates(x_ref[...], refs)
        h3_out[...] = dinv_ref[...] * h2

    cin, cout = lw['W_i'].shape[0], lw['Wg'].shape[1]
    return pl.pallas_call(
        body,
        grid=(NB,),
        in_specs=[
            pl.BlockSpec((R, cin), lambda i: (i, 0)),
            pl.BlockSpec((R, 1), lambda i: (i, 0)),
            *[_full(lw[n].shape) for n in names],
        ],
        out_specs=pl.BlockSpec((R, cout), lambda i: (i, 0)),
        out_shape=jax.ShapeDtypeStruct((N, cout), jnp.float32),
    )(x, dinv, *[lw[n] for n in names])


def _tc_post(agg2, h3, dinv, gb):
    def body(agg2_ref, h3_ref, dinv_ref, gb_ref, r_out, s1_out, s2_out):
        s = agg2_ref[0] + agg2_ref[1] + h3_ref[...]
        out = dinv_ref[...] * s + gb_ref[...]
        r = jnp.maximum(out, 0.0)
        r_out[...] = r

        @pl.when(pl.program_id(0) == 0)
        def _():
            s1_out[...] = jnp.zeros_like(s1_out)
            s2_out[...] = jnp.zeros_like(s2_out)

        s1_out[...] += jnp.sum(r, axis=0, keepdims=True)
        s2_out[...] += jnp.sum(r * r, axis=0, keepdims=True)

    cout = h3.shape[1]
    return pl.pallas_call(
        body,
        grid=(NB,),
        in_specs=[
            pl.BlockSpec((2, R, cout), lambda i: (0, i, 0)),
            pl.BlockSpec((R, cout), lambda i: (i, 0)),
            pl.BlockSpec((R, 1), lambda i: (i, 0)),
            _full((1, cout)),
        ],
        out_specs=(pl.BlockSpec((R, cout), lambda i: (i, 0)),
                   _full((1, cout)), _full((1, cout))),
        out_shape=(jax.ShapeDtypeStruct((N, cout), jnp.float32),
                   jax.ShapeDtypeStruct((1, cout), jnp.float32),
                   jax.ShapeDtypeStruct((1, cout), jnp.float32)),
    )(agg2, h3, dinv, gb)


def _bn(r_ref, s1_ref, s2_ref, bg_ref, bb_ref):
    mu = s1_ref[...] * (1.0 / N)
    var = s2_ref[...] * (1.0 / N) - mu * mu
    return ((r_ref[...] - mu) * _rsqrt(var + 1e-5) * bg_ref[...]
            + bb_ref[...])


def _tc_mid(r, s1, s2, dinv, bg, bb, lw):
    names = ['W_i', 'W_c', 'W_o', 'bi', 'bc', 'bo', 'wco', 'Wg']

    def body(r_ref, s1_ref, s2_ref, dinv_ref, bg_ref, bb_ref, *rest):
        refs = dict(zip(names, rest[:8]))
        h3_out = rest[8]
        hn = _bn(r_ref, s1_ref, s2_ref, bg_ref, bb_ref)
        h2 = _gates(hn, refs)
        h3_out[...] = dinv_ref[...] * h2

    cin, cout = lw['Wg'].shape[0], lw['Wg'].shape[1]
    return pl.pallas_call(
        body,
        grid=(NB,),
        in_specs=[
            pl.BlockSpec((R, r.shape[1]), lambda i: (i, 0)),
            _full(s1.shape), _full(s2.shape),
            pl.BlockSpec((R, 1), lambda i: (i, 0)),
            _full(bg.shape), _full(bb.shape),
            *[_full(lw[n].shape) for n in names],
        ],
        out_specs=pl.BlockSpec((R, cout), lambda i: (i, 0)),
        out_shape=jax.ShapeDtypeStruct((N, cout), jnp.float32),
    )(r, s1, s2, dinv, bg, bb, *[lw[n] for n in names])


def _tc_last(r, s1, s2, bg, bb, lin_W, lin_b):
    def body(r_ref, s1_ref, s2_ref, bg_ref, bb_ref, lw_ref, lb_ref, o_ref):
        hn = _bn(r_ref, s1_ref, s2_ref, bg_ref, bb_ref)
        o_ref[...] = _dot(hn, lw_ref[...]) + lb_ref[...]

    return pl.pallas_call(
        body,
        grid=(NB,),
        in_specs=[
            pl.BlockSpec((R, r.shape[1]), lambda i: (i, 0)),
            _full(s1.shape), _full(s2.shape),
            _full(bg.shape), _full(bb.shape),
            _full(lin_W.shape), _full((1, 1)),
        ],
        out_specs=pl.BlockSpec((R, 1), lambda i: (i, 0)),
        out_shape=jax.ShapeDtypeStruct((N, 1), jnp.float32),
    )(r, s1, s2, bg, bb, lin_W, lin_b.reshape(1, 1))


def _layer_weights(params, li):
    p = params['lstm%d' % li]
    return {
        'W_i': p['W_i'], 'W_c': p['W_c'], 'W_o': p['W_o'],
        'bi': (p['conv_i_bias'] + p['b_i'][0]).reshape(1, -1),
        'bc': (p['conv_c_bias'] + p['b_c'][0]).reshape(1, -1),
        'bo': (p['conv_o_bias'] + p['b_o'][0]).reshape(1, -1),
        'wco': p['w_c_o'],
        'Wg': params['gcn%d_W' % li],
    }


def kernel(x, edge_index, edge_weight, params):
    del edge_weight
    rows3 = edge_index[0].reshape(NW, KCH, CHUNK)
    cols3 = edge_index[1].reshape(NW, KCH, CHUNK)

    ones16 = jnp.ones((N, 16), jnp.float32)
    deg2 = _sc_aggregate(ones16, rows3, cols3, 16)
    deg = deg2[0, :, 0:1] + deg2[1, :, 0:1] + 1.0
    dinv = deg ** -0.5

    h3 = _tc_first(x, dinv, _layer_weights(params, 0))
    for li in range(4):
        agg2 = _sc_aggregate(h3, rows3, cols3, h3.shape[1])
        r, s1, s2 = _tc_post(agg2, h3, dinv,
                             params['gcn%d_b' % li].reshape(1, -1))
        bg = params['bn%d_g' % li].reshape(1, -1)
        bb = params['bn%d_b' % li].reshape(1, -1)
        if li < 3:
            h3 = _tc_mid(r, s1, s2, dinv, bg, bb,
                         _layer_weights(params, li + 1))
        else:
            return _tc_last(r, s1, s2, bg, bb,
                            params['lin_W'], params['lin_b'])
